# Initial kernel scaffold; baseline (speedup 1.0000x reference)
import jax
import jax.numpy as jnp
from jax.experimental import pallas as pl


def kernel(x, table, W, b, bn_gamma, bn_beta, bn_mean, bn_var, ln_gamma, ln_beta):
    raise NotImplementedError("write your pallas kernel here")


# speedup vs baseline: 11.0776x; 11.0776x over previous
import functools

import jax
import jax.numpy as jnp
from jax import lax
from jax.experimental import pallas as pl
from jax.experimental.pallas import tpu as pltpu
from jax.experimental.pallas import tpu_sc as plsc

B = 4096
SEQ = 50
D = 128
BN_EPS = 1e-3
LN_EPS = 1e-3

NC = 2
NS = 16
NW = NC * NS
BPW = B // NW
CHUNK_IDX = 100
ROWS_PER_CHUNK = CHUNK_IDX // SEQ
NCHUNK = BPW // ROWS_PER_CHUNK
NVEC = D // 16


def _sc_pool_body(xr_hbm, table_hbm, out_hbm, idx_v, rows_v, out_v, sem0, sem1):
    wid = lax.axis_index("s") * NC + lax.axis_index("c")

    pltpu.sync_copy(xr_hbm.at[pl.ds(wid * NCHUNK, NCHUNK)], idx_v)

    buf0 = rows_v.at[0]
    buf1 = rows_v.at[1]

    pltpu.async_copy(table_hbm.at[idx_v.at[0]], buf0, sem0)
    pltpu.async_copy(table_hbm.at[idx_v.at[1]], buf1, sem1)

    def accumulate(buf, out_row0):
        def jbody(j, acc):
            new0 = tuple(acc[d] + buf[j, pl.ds(16 * d, 16)] for d in range(NVEC))
            new1 = tuple(acc[NVEC + d] + buf[SEQ + j, pl.ds(16 * d, 16)]
                         for d in range(NVEC))
            return new0 + new1

        zero = tuple(jnp.zeros((16,), jnp.float32) for _ in range(2 * NVEC))
        acc = lax.fori_loop(0, SEQ, jbody, zero)
        for d in range(NVEC):
            out_v[out_row0, pl.ds(16 * d, 16)] = acc[d]
            out_v[out_row0 + 1, pl.ds(16 * d, 16)] = acc[NVEC + d]

    def ibody(i, carry):
        g = 2 * i
        pltpu.make_async_copy(table_hbm.at[idx_v.at[g]], buf0, sem0).wait()
        accumulate(buf0, 2 * g)

        @pl.when(i < NCHUNK // 2 - 1)
        def _():
            pltpu.async_copy(table_hbm.at[idx_v.at[g + 2]], buf0, sem0)

        pltpu.make_async_copy(table_hbm.at[idx_v.at[g + 1]], buf1, sem1).wait()
        accumulate(buf1, 2 * (g + 1))

        @pl.when(i < NCHUNK // 2 - 1)
        def _():
            pltpu.async_copy(table_hbm.at[idx_v.at[g + 3]], buf1, sem1)

        return carry

    lax.fori_loop(0, NCHUNK // 2, ibody, 0)

    pltpu.sync_copy(out_v, out_hbm.at[pl.ds(wid * BPW, BPW)])


_sc_pool = functools.partial(
    pl.kernel,
    out_type=jax.ShapeDtypeStruct((B, D), jnp.float32),
    mesh=plsc.VectorSubcoreMesh(core_axis_name="c", subcore_axis_name="s"),
    scratch_types=[
        pltpu.VMEM((NW * NCHUNK // NW, CHUNK_IDX), jnp.int32),
        pltpu.VMEM((2, CHUNK_IDX, D), jnp.float32),
        pltpu.VMEM((BPW, D), jnp.float32),
        pltpu.SemaphoreType.DMA,
        pltpu.SemaphoreType.DMA,
    ],
)(_sc_pool_body)


def _tc_dense_body(p_ref, w_ref, b_ref, bg_ref, bb_ref, bm_ref, bv_ref,
                   lg_ref, lb_ref, o_ref):
    x = p_ref[...] * (1.0 / SEQ)
    h = jnp.dot(x, w_ref[...], preferred_element_type=jnp.float32) + b_ref[...]
    bn_scale = bg_ref[...] * lax.rsqrt(bv_ref[...] + BN_EPS)
    h = (h - bm_ref[...]) * bn_scale + bb_ref[...]
    mu = jnp.mean(h, axis=1, keepdims=True)
    hc = h - mu
    var = jnp.mean(hc * hc, axis=1, keepdims=True)
    o_ref[...] = hc * lax.rsqrt(var + LN_EPS) * lg_ref[...] + lb_ref[...]


def kernel(x, table, W, b, bn_gamma, bn_beta, bn_mean, bn_var, ln_gamma,
           ln_beta):
    assert x.shape == (B, SEQ) and table.shape[1] == D

    xr = x.astype(jnp.int32).reshape(B * SEQ // CHUNK_IDX, CHUNK_IDX)
    pooled_sum = _sc_pool(xr, table)

    vec = lambda v: v.astype(jnp.float32).reshape(1, D)
    out = pl.pallas_call(
        _tc_dense_body,
        out_shape=jax.ShapeDtypeStruct((B, D), jnp.float32),
    )(pooled_sum, W, vec(b), vec(bn_gamma), vec(bn_beta), vec(bn_mean),
      vec(bn_var), vec(ln_gamma), vec(ln_beta))
    return out


# speedup vs baseline: 14.4960x; 1.3086x over previous
import functools

import jax
import jax.numpy as jnp
from jax import lax
from jax.experimental import pallas as pl
from jax.experimental.pallas import tpu as pltpu
from jax.experimental.pallas import tpu_sc as plsc

B = 4096
SEQ = 50
D = 128
BN_EPS = 1e-3
LN_EPS = 1e-3

NC = 2
NS = 16
NW = NC * NS
BPW = B // NW
CHUNK_IDX = 100
NBUF = 4
ROWS_PER_CHUNK = CHUNK_IDX // SEQ
NCHUNK = BPW // ROWS_PER_CHUNK
NVEC = D // 16


def _sc_pool_body(xr_hbm, table_hbm, out_hbm, idx_v, rows_v, out_v,
                  sem0, sem1, sem2, sem3):
    wid = lax.axis_index("s") * NC + lax.axis_index("c")

    pltpu.sync_copy(xr_hbm.at[pl.ds(wid * NCHUNK, NCHUNK)], idx_v)

    bufs = [rows_v.at[k] for k in range(NBUF)]
    sems = [sem0, sem1, sem2, sem3]

    for k in range(NBUF):
        pltpu.async_copy(table_hbm.at[idx_v.at[k]], bufs[k], sems[k])

    def accumulate(buf, out_row0):
        def jbody(j2, acc):
            j = 2 * j2
            for jj in (j, j + 1):
                acc = tuple(acc[d] + buf[jj, pl.ds(16 * d, 16)]
                            for d in range(NVEC)) + tuple(
                    acc[NVEC + d] + buf[SEQ + jj, pl.ds(16 * d, 16)]
                    for d in range(NVEC))
            return acc

        zero = tuple(jnp.zeros((16,), jnp.float32) for _ in range(2 * NVEC))
        acc = lax.fori_loop(0, SEQ // 2, jbody, zero)
        for d in range(NVEC):
            out_v[out_row0, pl.ds(16 * d, 16)] = acc[d]
            out_v[out_row0 + 1, pl.ds(16 * d, 16)] = acc[NVEC + d]

    def ibody(i, carry):
        g = NBUF * i
        for k in range(NBUF):
            pltpu.make_async_copy(
                table_hbm.at[idx_v.at[g + k]], bufs[k], sems[k]).wait()
            accumulate(bufs[k], 2 * (g + k))

            @pl.when(i < NCHUNK // NBUF - 1)
            def _():
                pltpu.async_copy(
                    table_hbm.at[idx_v.at[g + k + NBUF]], bufs[k], sems[k])

        return carry

    lax.fori_loop(0, NCHUNK // NBUF, ibody, 0)

    pltpu.sync_copy(out_v, out_hbm.at[pl.ds(wid * BPW, BPW)])


_sc_pool = functools.partial(
    pl.kernel,
    out_type=jax.ShapeDtypeStruct((B, D), jnp.float32),
    mesh=plsc.VectorSubcoreMesh(core_axis_name="c", subcore_axis_name="s"),
    scratch_types=[
        pltpu.VMEM((NW * NCHUNK // NW, CHUNK_IDX), jnp.int32),
        pltpu.VMEM((NBUF, CHUNK_IDX, D), jnp.float32),
        pltpu.VMEM((BPW, D), jnp.float32),
        pltpu.SemaphoreType.DMA,
        pltpu.SemaphoreType.DMA,
        pltpu.SemaphoreType.DMA,
        pltpu.SemaphoreType.DMA,
    ],
)(_sc_pool_body)


def _tc_dense_body(p_ref, w_ref, b_ref, bg_ref, bb_ref, bm_ref, bv_ref,
                   lg_ref, lb_ref, o_ref):
    x = p_ref[...] * (1.0 / SEQ)
    h = jnp.dot(x, w_ref[...], preferred_element_type=jnp.float32) + b_ref[...]
    bn_scale = bg_ref[...] * lax.rsqrt(bv_ref[...] + BN_EPS)
    h = (h - bm_ref[...]) * bn_scale + bb_ref[...]
    mu = jnp.mean(h, axis=1, keepdims=True)
    hc = h - mu
    var = jnp.mean(hc * hc, axis=1, keepdims=True)
    o_ref[...] = hc * lax.rsqrt(var + LN_EPS) * lg_ref[...] + lb_ref[...]


def kernel(x, table, W, b, bn_gamma, bn_beta, bn_mean, bn_var, ln_gamma,
           ln_beta):
    assert x.shape == (B, SEQ) and table.shape[1] == D

    xr = x.astype(jnp.int32).reshape(B * SEQ // CHUNK_IDX, CHUNK_IDX)
    pooled_sum = _sc_pool(xr, table)

    vec = lambda v: v.astype(jnp.float32).reshape(1, D)
    out = pl.pallas_call(
        _tc_dense_body,
        out_shape=jax.ShapeDtypeStruct((B, D), jnp.float32),
    )(pooled_sum, W, vec(b), vec(bn_gamma), vec(bn_beta), vec(bn_mean),
      vec(bn_var), vec(ln_gamma), vec(ln_beta))
    return out
